# baseline (speedup 1.0000x reference)
import functools

import jax
import jax.numpy as jnp
from jax import lax
from jax.experimental import pallas as pl
from jax.experimental.pallas import tpu as pltpu
from jax.experimental.pallas import tpu_sc as plsc

_N_CODES = 8192
_DIM = 256
_RB = 512
_NI = 18
_CHUNK = 2736
_NJ = 3


def _round_bf16_f32(x):
    r = jax.lax.bitcast_convert_type(x, jnp.int32)
    odd = jax.lax.shift_right_logical(r, 16) & jnp.int32(1)
    rr = (r + jnp.int32(0x7FFF) + odd) & jnp.int32(-65536)
    return jax.lax.bitcast_convert_type(rr, jnp.float32)


def _dist_body(fnorm, flat2, cb0, cb1, cb2, cn0, cn1, cn2, codes, dmin):
    big = jnp.float32(2 * _N_CODES)
    run_v = None
    for c, (cb, cn) in enumerate(((cb0, cn0), (cb1, cn1), (cb2, cn2))):
        lo = c * _CHUNK
        hi = min((c + 1) * _CHUNK, _N_CODES)
        ab2 = jax.lax.dot_general(flat2[...], cb[...],
                                  (((1,), (1,)), ((), ())),
                                  preferred_element_type=jnp.float32)
        d2 = (fnorm[...] - ab2) + cn[...]
        sc = jnp.sqrt(jnp.maximum(d2, 0.0))
        colf = (jax.lax.broadcasted_iota(jnp.int32, (_RB, hi - lo), 1)
                .astype(jnp.float32))
        m = jnp.min(sc, axis=1, keepdims=True)
        idx = jnp.min(jnp.where(sc == m, colf + jnp.float32(lo), big),
                      axis=1, keepdims=True)
        if run_v is None:
            run_v = _round_bf16_f32(m)
            run_i, run_m = idx, m
        else:
            take = m < run_v
            run_v = jnp.where(take, _round_bf16_f32(m), run_v)
            run_i = jnp.where(take, idx, run_i)
            run_m = jnp.where(take, m, run_m)
    codes[...] = run_i.astype(jnp.int32)
    dmin[...] = run_m


def _distance_argmin(fnorm, flat, cbs, cns):
    cb_specs = [pl.BlockSpec(c.shape, lambda i: (0, 0)) for c in cbs]
    cn_specs = [pl.BlockSpec(c.shape, lambda i: (0, 0)) for c in cns]
    return pl.pallas_call(
        _dist_body,
        grid=(_NI,),
        in_specs=[
            pl.BlockSpec((_RB, 1), lambda i: (i, 0)),
            pl.BlockSpec((_RB, _DIM), lambda i: (i, 0)),
            *cb_specs,
            *cn_specs,
        ],
        out_specs=[
            pl.BlockSpec((_RB, 1), lambda i: (i, 0)),
            pl.BlockSpec((_RB, 1), lambda i: (i, 0)),
        ],
        out_shape=[
            jax.ShapeDtypeStruct((_NI * _RB, 1), jnp.int32),
            jax.ShapeDtypeStruct((_NI * _RB, 1), jnp.float32),
        ],
    )(fnorm, flat, *cbs, *cns)


_NW = 32
_BPW = 9216 // _NW
_SUB = 96
_NSUB = _BPW // _SUB


def _sc_gather_hist(codebook, codes_flat, zeros8k, ones96):
    mesh = plsc.VectorSubcoreMesh(core_axis_name="c", subcore_axis_name="s")

    @functools.partial(
        pl.kernel, mesh=mesh,
        out_type=[jax.ShapeDtypeStruct((9216, _DIM), jnp.float32),
                  jax.ShapeDtypeStruct((2, _N_CODES), jnp.float32)],
        scratch_types=[
            pltpu.VMEM((_NSUB, _SUB), jnp.int32),
            pltpu.VMEM((_SUB, _DIM), jnp.float32),
            pltpu.VMEM((_SUB,), jnp.float32),
            pltpu.VMEM_SHARED((_N_CODES,), jnp.float32),
            pltpu.SemaphoreType.DMA,
        ])
    def k(cb_hbm, codes_hbm, z_hbm, one_hbm, quant_hbm, counts_hbm,
          idx_v, rows_v, ones_v, shared, sem):
        cid = lax.axis_index("c")
        sid = lax.axis_index("s")
        wid = sid * 2 + cid
        base = wid * _BPW
        for c in range(_NSUB):
            pltpu.sync_copy(codes_hbm.at[pl.ds(base + c * _SUB, _SUB)],
                            idx_v.at[c])
            pltpu.async_copy(cb_hbm.at[idx_v.at[c]], rows_v, sem).wait()
            pltpu.sync_copy(rows_v, quant_hbm.at[pl.ds(base + c * _SUB, _SUB)])
        pltpu.sync_copy(one_hbm, ones_v)

        @pl.when(sid == 0)
        def _():
            pltpu.sync_copy(z_hbm, shared)

        plsc.subcore_barrier()
        for c in range(_NSUB):
            pltpu.sync_copy(ones_v, shared.at[idx_v.at[c]], add=True)
        plsc.subcore_barrier()

        @pl.when(sid == 0)
        def _():
            pltpu.sync_copy(shared, counts_hbm.at[cid])

    return k(codebook, codes_flat, zeros8k, ones96)


def kernel(latents, codebook):
    shape = latents.shape
    flat = latents.reshape(-1, _DIM)
    flat2 = (2.0 * latents).astype(jnp.bfloat16).reshape(-1, _DIM)
    fnorm = jnp.sum(latents * latents, axis=-1).reshape(-1, 1)
    cnorm = jnp.sum(codebook * codebook, axis=1)[None, :]
    cbs = tuple(codebook[c * _CHUNK:min((c + 1) * _CHUNK, _N_CODES)]
                for c in range(_NJ))
    cns = tuple(cnorm[:, c * _CHUNK:min((c + 1) * _CHUNK, _N_CODES)]
                for c in range(_NJ))
    codes2, dmin = _distance_argmin(fnorm, flat2, cbs, cns)
    codes_flat = codes2.reshape(-1)
    codes = codes_flat.reshape(shape[:-1])
    quant, counts2 = _sc_gather_hist(
        codebook, codes_flat, jnp.zeros((_N_CODES,), jnp.float32),
        jnp.ones((_SUB,), jnp.float32))
    quantized = quant.reshape(shape)
    codebook_loss = jnp.sum(dmin * dmin) / jnp.float32(9216 * _DIM)
    commitment_loss = codebook_loss
    quantized_st = latents + jax.lax.stop_gradient(quantized - latents)
    counts = counts2[0] + counts2[1]
    probs = counts / jnp.sum(counts)
    entropy = -jnp.sum(probs * jnp.log(probs + 1e-10))
    perplexity = jnp.exp(entropy)
    return (quantized_st, codes, commitment_loss * 0.25, codebook_loss,
            perplexity)


# speedup vs baseline: 1.0164x; 1.0164x over previous
import functools

import jax
import jax.numpy as jnp
from jax import lax
from jax.experimental import pallas as pl
from jax.experimental.pallas import tpu as pltpu
from jax.experimental.pallas import tpu_sc as plsc

_N_CODES = 8192
_DIM = 256
_RB = 512
_NI = 18
_CHUNK = 2736
_NJ = 3


def _round_bf16_f32(x):
    r = jax.lax.bitcast_convert_type(x, jnp.int32)
    odd = jax.lax.shift_right_logical(r, 16) & jnp.int32(1)
    rr = (r + jnp.int32(0x7FFF) + odd) & jnp.int32(-65536)
    return jax.lax.bitcast_convert_type(rr, jnp.float32)


def _dist_body(fnorm, flat2, cb, cnorm, codes, dmin):
    ab2 = jax.lax.dot_general(flat2[...], cb[...], (((1,), (1,)), ((), ())),
                              preferred_element_type=jnp.float32)
    d2 = (fnorm[...] - ab2) + cnorm[...]
    s = jnp.sqrt(jnp.maximum(d2, 0.0))
    big = jnp.float32(2 * _N_CODES)
    run_v = None
    for c in range(_NJ):
        lo = c * _CHUNK
        hi = min((c + 1) * _CHUNK, _N_CODES)
        sc = jax.lax.slice(s, (0, lo), (_RB, hi))
        colf = (jax.lax.broadcasted_iota(jnp.int32, (_RB, hi - lo), 1)
                .astype(jnp.float32))
        m = jnp.min(sc, axis=1, keepdims=True)
        idx = jnp.min(jnp.where(sc == m, colf + jnp.float32(lo), big),
                      axis=1, keepdims=True)
        if run_v is None:
            run_v = _round_bf16_f32(m)
            run_i, run_m = idx, m
        else:
            take = m < run_v
            run_v = jnp.where(take, _round_bf16_f32(m), run_v)
            run_i = jnp.where(take, idx, run_i)
            run_m = jnp.where(take, m, run_m)
    codes[...] = run_i.astype(jnp.int32)
    dmin[...] = run_m


def _distance_argmin(fnorm, flat, codebook, cnorm):
    return pl.pallas_call(
        _dist_body,
        grid=(_NI,),
        in_specs=[
            pl.BlockSpec((_RB, 1), lambda i: (i, 0)),
            pl.BlockSpec((_RB, _DIM), lambda i: (i, 0)),
            pl.BlockSpec((_N_CODES, _DIM), lambda i: (0, 0)),
            pl.BlockSpec((1, _N_CODES), lambda i: (0, 0)),
        ],
        out_specs=[
            pl.BlockSpec((_RB, 1), lambda i: (i, 0)),
            pl.BlockSpec((_RB, 1), lambda i: (i, 0)),
        ],
        out_shape=[
            jax.ShapeDtypeStruct((_NI * _RB, 1), jnp.int32),
            jax.ShapeDtypeStruct((_NI * _RB, 1), jnp.float32),
        ],
    )(fnorm, flat, codebook, cnorm)


_NW = 32
_BPW = 9216 // _NW
_SUB = 96
_NSUB = _BPW // _SUB


def _sc_gather_hist(codebook, codes_flat, zeros8k, ones96):
    mesh = plsc.VectorSubcoreMesh(core_axis_name="c", subcore_axis_name="s")

    @functools.partial(
        pl.kernel, mesh=mesh,
        out_type=[jax.ShapeDtypeStruct((9216, _DIM), jnp.float32),
                  jax.ShapeDtypeStruct((2, _N_CODES), jnp.float32)],
        scratch_types=[
            pltpu.VMEM((_NSUB, _SUB), jnp.int32),
            pltpu.VMEM((_SUB, _DIM), jnp.float32),
            pltpu.VMEM((_SUB,), jnp.float32),
            pltpu.VMEM_SHARED((_N_CODES,), jnp.float32),
            pltpu.SemaphoreType.DMA,
        ])
    def k(cb_hbm, codes_hbm, z_hbm, one_hbm, quant_hbm, counts_hbm,
          idx_v, rows_v, ones_v, shared, sem):
        cid = lax.axis_index("c")
        sid = lax.axis_index("s")
        wid = sid * 2 + cid
        base = wid * _BPW
        for c in range(_NSUB):
            pltpu.sync_copy(codes_hbm.at[pl.ds(base + c * _SUB, _SUB)],
                            idx_v.at[c])
            pltpu.async_copy(cb_hbm.at[idx_v.at[c]], rows_v, sem).wait()
            pltpu.sync_copy(rows_v, quant_hbm.at[pl.ds(base + c * _SUB, _SUB)])
        pltpu.sync_copy(one_hbm, ones_v)

        @pl.when(sid == 0)
        def _():
            pltpu.sync_copy(z_hbm, shared)

        plsc.subcore_barrier()
        for c in range(_NSUB):
            pltpu.sync_copy(ones_v, shared.at[idx_v.at[c]], add=True)
        plsc.subcore_barrier()

        @pl.when(sid == 0)
        def _():
            pltpu.sync_copy(shared, counts_hbm.at[cid])

    return k(codebook, codes_flat, zeros8k, ones96)


def kernel(latents, codebook):
    shape = latents.shape
    flat = latents.reshape(-1, _DIM)
    flat2 = (2.0 * latents).astype(jnp.bfloat16).reshape(-1, _DIM)
    fnorm = jnp.sum(latents * latents, axis=-1).reshape(-1, 1)
    cnorm = jnp.sum(codebook * codebook, axis=1)[None, :]
    codes2, dmin = _distance_argmin(fnorm, flat2, codebook, cnorm)
    codes_flat = codes2.reshape(-1)
    codes = codes_flat.reshape(shape[:-1])
    quant, counts2 = _sc_gather_hist(
        codebook, codes_flat, jnp.zeros((_N_CODES,), jnp.float32),
        jnp.ones((_SUB,), jnp.float32))
    quantized = quant.reshape(shape)
    codebook_loss = jnp.sum(dmin * dmin) / jnp.float32(9216 * _DIM)
    commitment_loss = codebook_loss
    quantized_st = latents + jax.lax.stop_gradient(quantized - latents)
    counts = counts2[0] + counts2[1]
    probs = counts / jnp.sum(counts)
    entropy = -jnp.sum(probs * jnp.log(probs + 1e-10))
    perplexity = jnp.exp(entropy)
    return (quantized_st, codes, commitment_loss * 0.25, codebook_loss,
            perplexity)


# speedup vs baseline: 1.0845x; 1.0670x over previous
import functools

import jax
import jax.numpy as jnp
from jax import lax
from jax.experimental import pallas as pl
from jax.experimental.pallas import tpu as pltpu
from jax.experimental.pallas import tpu_sc as plsc

_N_CODES = 8192
_DIM = 256
_RB = 1024
_NI = 9
_CHUNK = 2736
_NJ = 3


def _round_bf16_f32(x):
    r = jax.lax.bitcast_convert_type(x, jnp.int32)
    odd = jax.lax.shift_right_logical(r, 16) & jnp.int32(1)
    rr = (r + jnp.int32(0x7FFF) + odd) & jnp.int32(-65536)
    return jax.lax.bitcast_convert_type(rr, jnp.float32)


def _dist_body(fnorm, flat2, cb, cnorm, codes, dmin):
    ab2 = jax.lax.dot_general(flat2[...], cb[...], (((1,), (1,)), ((), ())),
                              preferred_element_type=jnp.float32)
    d2 = (fnorm[...] - ab2) + cnorm[...]
    s = jnp.sqrt(jnp.maximum(d2, 0.0))
    big = jnp.float32(2 * _N_CODES)
    run_v = None
    for c in range(_NJ):
        lo = c * _CHUNK
        hi = min((c + 1) * _CHUNK, _N_CODES)
        sc = jax.lax.slice(s, (0, lo), (_RB, hi))
        colf = (jax.lax.broadcasted_iota(jnp.int32, (_RB, hi - lo), 1)
                .astype(jnp.float32))
        m = jnp.min(sc, axis=1, keepdims=True)
        idx = jnp.min(jnp.where(sc == m, colf + jnp.float32(lo), big),
                      axis=1, keepdims=True)
        if run_v is None:
            run_v = _round_bf16_f32(m)
            run_i, run_m = idx, m
        else:
            take = m < run_v
            run_v = jnp.where(take, _round_bf16_f32(m), run_v)
            run_i = jnp.where(take, idx, run_i)
            run_m = jnp.where(take, m, run_m)
    codes[...] = run_i.astype(jnp.int32)
    dmin[...] = run_m


def _distance_argmin(fnorm, flat, codebook, cnorm):
    return pl.pallas_call(
        _dist_body,
        grid=(_NI,),
        in_specs=[
            pl.BlockSpec((_RB, 1), lambda i: (i, 0)),
            pl.BlockSpec((_RB, _DIM), lambda i: (i, 0)),
            pl.BlockSpec((_N_CODES, _DIM), lambda i: (0, 0)),
            pl.BlockSpec((1, _N_CODES), lambda i: (0, 0)),
        ],
        out_specs=[
            pl.BlockSpec((_RB, 1), lambda i: (i, 0)),
            pl.BlockSpec((_RB, 1), lambda i: (i, 0)),
        ],
        out_shape=[
            jax.ShapeDtypeStruct((_NI * _RB, 1), jnp.int32),
            jax.ShapeDtypeStruct((_NI * _RB, 1), jnp.float32),
        ],
    )(fnorm, flat, codebook, cnorm)


_NW = 32
_BPW = 9216 // _NW
_SUB = 96
_NSUB = _BPW // _SUB


def _sc_gather_hist(codebook, codes_flat, zeros8k, ones96):
    mesh = plsc.VectorSubcoreMesh(core_axis_name="c", subcore_axis_name="s")

    @functools.partial(
        pl.kernel, mesh=mesh,
        out_type=[jax.ShapeDtypeStruct((9216, _DIM), jnp.float32),
                  jax.ShapeDtypeStruct((2, _N_CODES), jnp.float32)],
        scratch_types=[
            pltpu.VMEM((_NSUB, _SUB), jnp.int32),
            pltpu.VMEM((_SUB, _DIM), jnp.float32),
            pltpu.VMEM((_SUB,), jnp.float32),
            pltpu.VMEM_SHARED((_N_CODES,), jnp.float32),
            pltpu.SemaphoreType.DMA,
        ])
    def k(cb_hbm, codes_hbm, z_hbm, one_hbm, quant_hbm, counts_hbm,
          idx_v, rows_v, ones_v, shared, sem):
        cid = lax.axis_index("c")
        sid = lax.axis_index("s")
        wid = sid * 2 + cid
        base = wid * _BPW
        for c in range(_NSUB):
            pltpu.sync_copy(codes_hbm.at[pl.ds(base + c * _SUB, _SUB)],
                            idx_v.at[c])
            pltpu.async_copy(cb_hbm.at[idx_v.at[c]], rows_v, sem).wait()
            pltpu.sync_copy(rows_v, quant_hbm.at[pl.ds(base + c * _SUB, _SUB)])
        pltpu.sync_copy(one_hbm, ones_v)

        @pl.when(sid == 0)
        def _():
            pltpu.sync_copy(z_hbm, shared)

        plsc.subcore_barrier()
        for c in range(_NSUB):
            pltpu.sync_copy(ones_v, shared.at[idx_v.at[c]], add=True)
        plsc.subcore_barrier()

        @pl.when(sid == 0)
        def _():
            pltpu.sync_copy(shared, counts_hbm.at[cid])

    return k(codebook, codes_flat, zeros8k, ones96)


def kernel(latents, codebook):
    shape = latents.shape
    flat2 = (2.0 * latents).astype(jnp.bfloat16).reshape(-1, _DIM)
    fnorm = jnp.sum(latents * latents, axis=-1).reshape(-1, 1)
    cnorm = jnp.sum(codebook * codebook, axis=1)[None, :]
    codes2, dmin = _distance_argmin(fnorm, flat2, codebook, cnorm)
    codes_flat = codes2.reshape(-1)
    codes = codes_flat.reshape(shape[:-1])
    quant, counts2 = _sc_gather_hist(
        codebook, codes_flat, jnp.zeros((_N_CODES,), jnp.float32),
        jnp.ones((_SUB,), jnp.float32))
    quantized = quant.reshape(shape)
    codebook_loss = jnp.sum(dmin * dmin) / jnp.float32(9216 * _DIM)
    commitment_loss = codebook_loss
    quantized_st = latents + jax.lax.stop_gradient(quantized - latents)
    counts = counts2[0] + counts2[1]
    probs = counts / jnp.sum(counts)
    entropy = -jnp.sum(probs * jnp.log(probs + 1e-10))
    perplexity = jnp.exp(entropy)
    return (quantized_st, codes, commitment_loss * 0.25, codebook_loss,
            perplexity)
